# baseline (speedup 1.0000x reference)
import functools

import jax
import jax.numpy as jnp
from jax import lax
from jax.experimental import pallas as pl
from jax.experimental.pallas import tpu as pltpu
from jax.experimental.pallas import tpu_sc as plsc

N_ATOMS = 10000
N_PROBES = 8000
N_EDGES = 160000
B = 4
P_MAX = 2000
F_HID = 128
D_RBF = 32
CUTOFF = 5.0

NC = 2
NS = 16
LANES = 16
NW = NC * NS

ROW = 4 * F_HID
E_PAD = 163840
EPW = E_PAD // NW
CH = 64
CHX = 1280
NCH = EPW // CHX
DUMP = 8191

P_PAD = 8192
SEG = P_PAD // NS

TBLK = 640
NH = 8
E_HALF = E_PAD // NH
EPW_G = E_HALF // NW
NTB = E_HALF // TBLK

_MESH = dict(core_axis_name="c", subcore_axis_name="s",
             num_cores=NC, num_subcores=NS)


def _gather_body(tbl_hbm, ea_hbm, outg_hbm, idx_v, rows_v, semg, sems):
    wid = lax.axis_index("s") * NC + lax.axis_index("c")
    base0 = wid * EPW_G
    nchk = EPW_G // CH
    pltpu.sync_copy(ea_hbm.at[pl.ds(base0, EPW_G)], idx_v)
    pltpu.async_copy(tbl_hbm.at[idx_v.at[pl.ds(0, CH)]], rows_v.at[0], semg)

    def rows(c, carry):
        slot = lax.rem(c, 2)
        nxt = lax.rem(c + 1, 2)
        pltpu.make_async_copy(
            tbl_hbm.at[idx_v.at[pl.ds(c * CH, CH)]],
            rows_v.at[slot], semg).wait()

        @pl.when(c + 1 < nchk)
        def _start_next():
            @pl.when(c >= 1)
            def _drain_prev_store():
                pltpu.make_async_copy(
                    rows_v.at[nxt],
                    outg_hbm.at[pl.ds(base0, CH)], sems).wait()
            pltpu.async_copy(
                tbl_hbm.at[idx_v.at[pl.ds((c + 1) * CH, CH)]],
                rows_v.at[nxt], semg)

        pltpu.async_copy(
            rows_v.at[slot], outg_hbm.at[pl.ds(base0 + c * CH, CH)], sems)
        return carry

    lax.fori_loop(0, nchk, rows, 0)
    for _ in range(2):
        pltpu.make_async_copy(
            rows_v.at[0], outg_hbm.at[pl.ds(base0, CH)], sems).wait()


@functools.lru_cache(maxsize=None)
def _gather_call():
    return pl.kernel(
        _gather_body,
        out_type=jax.ShapeDtypeStruct((E_HALF, ROW), jnp.float32),
        mesh=plsc.VectorSubcoreMesh(**_MESH),
        scratch_types=[
            pltpu.VMEM((EPW_G,), jnp.int32),
            pltpu.VMEM((2, CH, ROW), jnp.float32),
            pltpu.SemaphoreType.DMA,
            pltpu.SemaphoreType.DMA,
        ],
        compiler_params=pltpu.CompilerParams(needs_layout_passes=False),
    )


def _xyz_body(xyz6_hbm, ea_hbm, ep_hbm, outx_hbm,
              tabs_v, idxa_v, idxp_v, stage_v):
    wid = lax.axis_index("s") * NC + lax.axis_index("c")
    base0 = wid * EPW
    pltpu.sync_copy(xyz6_hbm, tabs_v)

    def superchunk(cx, carry):
        base = base0 + cx * CHX
        pltpu.sync_copy(ea_hbm.at[pl.ds(base, CHX)], idxa_v)
        pltpu.sync_copy(ep_hbm.at[pl.ds(base, CHX)], idxp_v)

        def xyz(g, carry2):
            off = g * LANES
            ia = idxa_v[pl.ds(off, LANES)]
            ip = idxp_v[pl.ds(off, LANES)]
            for k in range(3):
                stage_v[k, pl.ds(off, LANES)] = plsc.load_gather(
                    tabs_v.at[k], [ia])
                stage_v[3 + k, pl.ds(off, LANES)] = plsc.load_gather(
                    tabs_v.at[3 + k], [ip])
            return carry2

        lax.fori_loop(0, CHX // LANES, xyz, 0)
        for k in range(6):
            pltpu.sync_copy(stage_v.at[k], outx_hbm.at[k, pl.ds(base, CHX)])
        return carry

    lax.fori_loop(0, NCH, superchunk, 0)


@functools.lru_cache(maxsize=None)
def _xyz_call():
    return pl.kernel(
        _xyz_body,
        out_type=jax.ShapeDtypeStruct((8, E_PAD), jnp.float32),
        mesh=plsc.VectorSubcoreMesh(**_MESH),
        scratch_types=[
            pltpu.VMEM((6, N_ATOMS), jnp.float32),
            pltpu.VMEM((CHX,), jnp.int32),
            pltpu.VMEM((CHX,), jnp.int32),
            pltpu.VMEM((6, CHX), jnp.float32),
        ],
        compiler_params=pltpu.CompilerParams(
            use_tc_tiling_on_sc=False, needs_layout_passes=False),
    )


def _scatter_body(mw_hbm, ep_hbm, out_hbm, mw_v, ep_v, rho_v, red_v, acc_v, shared):
    cid = lax.axis_index("c")
    sid = lax.axis_index("s")
    wid = sid * NC + cid
    base = wid * EPW
    pltpu.sync_copy(mw_hbm.at[pl.ds(base, EPW)], mw_v)
    pltpu.sync_copy(ep_hbm.at[pl.ds(base, EPW)], ep_v)

    def zero(j, carry):
        rho_v[pl.ds(j * LANES, LANES)] = jnp.zeros((LANES,), jnp.float32)
        return carry

    lax.fori_loop(0, P_PAD // LANES, zero, 0)

    def scat(i, carry):
        idx = ep_v[pl.ds(i * LANES, LANES)]
        val = mw_v[pl.ds(i * LANES, LANES)]
        plsc.addupdate_scatter(rho_v, [idx], val)
        return carry

    lax.fori_loop(0, EPW // LANES, scat, 0)

    pltpu.sync_copy(rho_v, shared.at[sid])
    plsc.subcore_barrier()

    def rd(k, carry):
        pltpu.sync_copy(shared.at[k, pl.ds(sid * SEG, SEG)], red_v.at[k])
        return carry

    lax.fori_loop(0, NS, rd, 0)

    def red(j, carry):
        s = red_v[0, pl.ds(j * LANES, LANES)]
        for k in range(1, NS):
            s = s + red_v[k, pl.ds(j * LANES, LANES)]
        acc_v[pl.ds(j * LANES, LANES)] = s
        return carry

    lax.fori_loop(0, SEG // LANES, red, 0)
    pltpu.sync_copy(acc_v, out_hbm.at[cid, pl.ds(sid * SEG, SEG)])


@functools.lru_cache(maxsize=None)
def _scatter_call():
    return pl.kernel(
        _scatter_body,
        out_type=jax.ShapeDtypeStruct((NC, P_PAD), jnp.float32),
        mesh=plsc.VectorSubcoreMesh(**_MESH),
        scratch_types=[
            pltpu.VMEM((EPW,), jnp.float32),
            pltpu.VMEM((EPW,), jnp.int32),
            pltpu.VMEM((P_PAD,), jnp.float32),
            pltpu.VMEM((NS, SEG), jnp.float32),
            pltpu.VMEM((SEG,), jnp.float32),
            pltpu.VMEM_SHARED((NS, P_PAD), jnp.float32),
        ],
        compiler_params=pltpu.CompilerParams(
            use_tc_tiling_on_sc=False, needs_layout_passes=False),
    )


def _mlp_body(g_ref, x_ref, pk_ref, cmt_ref, w1a_ref, w1b_ref, w1c_ref,
              w1d_ref, b1_ref, w2_ref, b2_ref, w3_ref, b3_ref, out_ref):
    g = g_ref[...]
    s_e = g[:, 0:F_HID]
    v0 = g[:, F_HID:2 * F_HID]
    v1 = g[:, 2 * F_HID:3 * F_HID]
    v2 = g[:, 3 * F_HID:4 * F_HID]
    axt = x_ref[0:3, :]
    pxt = x_ref[3:6, :]
    dispt = pk_ref[0:3, :]
    pft = pk_ref[3:4, :]

    dot = functools.partial(jnp.dot, preferred_element_type=jnp.float32)
    bft = jnp.floor(pft * (1.0 / P_MAX))
    iota4 = lax.broadcasted_iota(jnp.int32, (B, 1), 0).astype(jnp.float32)
    oht = (bft == iota4).astype(jnp.float32)
    dvallt = jnp.dot(cmt_ref[0:3 * B, 0:3], dispt,
                     preferred_element_type=jnp.float32,
                     precision=lax.Precision.HIGHEST)
    dvt = (oht[0:1] * dvallt[0:3] + oht[1:2] * dvallt[3:6] +
           oht[2:3] * dvallt[6:9] + oht[3:4] * dvallt[9:12])
    difft = pxt - (axt + dvt)
    d2t = jnp.sum(difft * difft, axis=0, keepdims=True)
    distt = jnp.sqrt(d2t)
    invt = 1.0 / (distt + 1e-8)
    inv0t = 1.0 / distt
    cwt = 0.5 * (jnp.cos(distt * (jnp.pi / CUTOFF)) + 1.0)
    cwt = jnp.where(distt < CUTOFF, cwt, 0.0)
    nvec = lax.broadcasted_iota(jnp.int32, (D_RBF, 1), 0).astype(
        jnp.float32) + 1.0
    e_pit = jnp.sin(distt * nvec * (jnp.pi / CUTOFF)) * inv0t
    e_pi = jnp.transpose(e_pit)
    rhat = jnp.transpose(difft * invt)
    q_pi = v0 * rhat[:, 0:1] + v1 * rhat[:, 1:2] + v2 * rhat[:, 2:3]
    n_e = jnp.sqrt(v0 * v0 + v1 * v1 + v2 * v2)

    pre1 = (dot(e_pi, w1a_ref[...]) + dot(q_pi, w1b_ref[...]) +
            dot(n_e, w1c_ref[...]) + dot(s_e, w1d_ref[...]) + b1_ref[...])
    h1 = pre1 * jax.nn.sigmoid(pre1)
    pre2 = dot(h1, w2_ref[...]) + b2_ref[...]
    h2 = pre2 * jax.nn.sigmoid(pre2)
    m = jnp.sum(h2 * w3_ref[...], axis=1, keepdims=True) + b3_ref[...]
    out_ref[...] = m * jnp.transpose(cwt)


_mlp = pl.pallas_call(
    _mlp_body,
    grid=(NTB,),
    in_specs=[
        pl.BlockSpec((TBLK, ROW), lambda i: (i, 0)),
        pl.BlockSpec((8, TBLK), lambda i: (0, i)),
        pl.BlockSpec((8, TBLK), lambda i: (0, i)),
        pl.BlockSpec((16, 8), lambda i: (0, 0)),
        pl.BlockSpec((D_RBF, F_HID), lambda i: (0, 0)),
        pl.BlockSpec((F_HID, F_HID), lambda i: (0, 0)),
        pl.BlockSpec((F_HID, F_HID), lambda i: (0, 0)),
        pl.BlockSpec((F_HID, F_HID), lambda i: (0, 0)),
        pl.BlockSpec((1, F_HID), lambda i: (0, 0)),
        pl.BlockSpec((F_HID, F_HID // 2), lambda i: (0, 0)),
        pl.BlockSpec((1, F_HID // 2), lambda i: (0, 0)),
        pl.BlockSpec((1, F_HID // 2), lambda i: (0, 0)),
        pl.BlockSpec((1, 1), lambda i: (0, 0)),
    ],
    out_specs=pl.BlockSpec((TBLK, 1), lambda i: (i, 0)),
    out_shape=jax.ShapeDtypeStruct((E_HALF, 1), jnp.float32),
    compiler_params=pltpu.CompilerParams(
        dimension_semantics=("arbitrary",)),
)


def kernel(atom_xyz, probe_xyz, cell, edge_atom, edge_probe,
           probe_edges_displacement, S_JK, V_JK, W1, b1, W2, b2, W3, b3):
    f32 = jnp.float32
    tbl = jnp.concatenate(
        [S_JK, V_JK[:, 0, :], V_JK[:, 1, :], V_JK[:, 2, :]], axis=1)
    xyz6 = jnp.zeros((6, N_ATOMS), f32)
    xyz6 = xyz6.at[0:3, :].set(jnp.transpose(atom_xyz))
    xyz6 = xyz6.at[3:6, 0:N_PROBES].set(jnp.transpose(probe_xyz))

    npad = E_PAD - N_EDGES
    ea_pad = jnp.concatenate([edge_atom, jnp.zeros((npad,), jnp.int32)])
    ep_pad = jnp.concatenate([edge_probe, jnp.zeros((npad,), jnp.int32)])
    xyzt = _xyz_call()(xyz6, ea_pad, ep_pad)

    packedt = jnp.zeros((8, E_PAD), f32)
    packedt = packedt.at[0:3, 0:N_EDGES].set(
        jnp.transpose(probe_edges_displacement))
    packedt = packedt.at[3, 0:N_EDGES].set(edge_probe.astype(f32))
    cmt = jnp.zeros((16, 8), f32).at[:3 * B, :3].set(
        jnp.transpose(cell, (0, 2, 1)).reshape(3 * B, 3))
    mw_halves = []
    for h in range(NH):
        lo = h * E_HALF
        g_h = _gather_call()(tbl, lax.slice_in_dim(ea_pad, lo, lo + E_HALF))
        mw_h = _mlp(g_h, lax.slice_in_dim(xyzt, lo, lo + E_HALF, axis=1),
                    lax.slice_in_dim(packedt, lo, lo + E_HALF, axis=1), cmt,
                    W1[0:D_RBF], W1[D_RBF:D_RBF + F_HID],
                    W1[D_RBF + F_HID:D_RBF + 2 * F_HID],
                    W1[D_RBF + 2 * F_HID:D_RBF + 3 * F_HID],
                    b1[None, :], W2, b2[None, :],
                    W3.reshape(1, F_HID // 2), b3.reshape(1, 1))
        mw_halves.append(mw_h[:, 0])
    mw = jnp.concatenate(mw_halves)

    ep_scat = jnp.concatenate(
        [edge_probe, jnp.full((npad,), DUMP, jnp.int32)])
    part = _scatter_call()(mw, ep_scat)
    rho = part[0, :N_PROBES] + part[1, :N_PROBES]
    return rho.reshape(B, P_MAX)


# speedup vs baseline: 1.1231x; 1.1231x over previous
import functools

import jax
import jax.numpy as jnp
from jax import lax
from jax.experimental import pallas as pl
from jax.experimental.pallas import tpu as pltpu
from jax.experimental.pallas import tpu_sc as plsc

N_ATOMS = 10000
N_PROBES = 8000
N_EDGES = 160000
B = 4
P_MAX = 2000
F_HID = 128
D_RBF = 32
CUTOFF = 5.0

NC = 2
NS = 16
LANES = 16
NW = NC * NS

ROW = 4 * F_HID
E_PAD = 163840
EPW = E_PAD // NW
CH = 64
CHX = 1280
NCH = EPW // CHX
DUMP = 8191

P_PAD = 8192
SEG = P_PAD // NS

TBLK = 1280
NH = 8
E_HALF = E_PAD // NH
EPW_G = E_HALF // NW
NTB = E_HALF // TBLK

_MESH = dict(core_axis_name="c", subcore_axis_name="s",
             num_cores=NC, num_subcores=NS)


def _gather_body(tbl_hbm, ea_hbm, outg_hbm, idx_v, rows_v, semg, sems):
    wid = lax.axis_index("s") * NC + lax.axis_index("c")
    base0 = wid * EPW_G
    nchk = EPW_G // CH
    pltpu.sync_copy(ea_hbm.at[pl.ds(base0, EPW_G)], idx_v)
    pltpu.async_copy(tbl_hbm.at[idx_v.at[pl.ds(0, CH)]], rows_v.at[0], semg)

    def rows(c, carry):
        slot = lax.rem(c, 2)
        nxt = lax.rem(c + 1, 2)
        pltpu.make_async_copy(
            tbl_hbm.at[idx_v.at[pl.ds(c * CH, CH)]],
            rows_v.at[slot], semg).wait()

        @pl.when(c + 1 < nchk)
        def _start_next():
            @pl.when(c >= 1)
            def _drain_prev_store():
                pltpu.make_async_copy(
                    rows_v.at[nxt],
                    outg_hbm.at[pl.ds(base0, CH)], sems).wait()
            pltpu.async_copy(
                tbl_hbm.at[idx_v.at[pl.ds((c + 1) * CH, CH)]],
                rows_v.at[nxt], semg)

        pltpu.async_copy(
            rows_v.at[slot], outg_hbm.at[pl.ds(base0 + c * CH, CH)], sems)
        return carry

    lax.fori_loop(0, nchk, rows, 0)
    for _ in range(2):
        pltpu.make_async_copy(
            rows_v.at[0], outg_hbm.at[pl.ds(base0, CH)], sems).wait()


@functools.lru_cache(maxsize=None)
def _gather_call():
    return pl.kernel(
        _gather_body,
        out_type=jax.ShapeDtypeStruct((E_HALF, ROW), jnp.float32),
        mesh=plsc.VectorSubcoreMesh(**_MESH),
        scratch_types=[
            pltpu.VMEM((EPW_G,), jnp.int32),
            pltpu.VMEM((2, CH, ROW), jnp.float32),
            pltpu.SemaphoreType.DMA,
            pltpu.SemaphoreType.DMA,
        ],
        compiler_params=pltpu.CompilerParams(needs_layout_passes=False),
    )


def _xyz_body(xyz6_hbm, ea_hbm, ep_hbm, outx_hbm,
              tabs_v, idxa_v, idxp_v, stage_v):
    wid = lax.axis_index("s") * NC + lax.axis_index("c")
    base0 = wid * EPW
    pltpu.sync_copy(xyz6_hbm, tabs_v)

    def superchunk(cx, carry):
        base = base0 + cx * CHX
        pltpu.sync_copy(ea_hbm.at[pl.ds(base, CHX)], idxa_v)
        pltpu.sync_copy(ep_hbm.at[pl.ds(base, CHX)], idxp_v)

        def xyz(g, carry2):
            off = g * LANES
            ia = idxa_v[pl.ds(off, LANES)]
            ip = idxp_v[pl.ds(off, LANES)]
            for k in range(3):
                stage_v[k, pl.ds(off, LANES)] = plsc.load_gather(
                    tabs_v.at[k], [ia])
                stage_v[3 + k, pl.ds(off, LANES)] = plsc.load_gather(
                    tabs_v.at[3 + k], [ip])
            return carry2

        lax.fori_loop(0, CHX // LANES, xyz, 0)
        for k in range(6):
            pltpu.sync_copy(stage_v.at[k], outx_hbm.at[k, pl.ds(base, CHX)])
        return carry

    lax.fori_loop(0, NCH, superchunk, 0)


@functools.lru_cache(maxsize=None)
def _xyz_call():
    return pl.kernel(
        _xyz_body,
        out_type=jax.ShapeDtypeStruct((8, E_PAD), jnp.float32),
        mesh=plsc.VectorSubcoreMesh(**_MESH),
        scratch_types=[
            pltpu.VMEM((6, N_ATOMS), jnp.float32),
            pltpu.VMEM((CHX,), jnp.int32),
            pltpu.VMEM((CHX,), jnp.int32),
            pltpu.VMEM((6, CHX), jnp.float32),
        ],
        compiler_params=pltpu.CompilerParams(
            use_tc_tiling_on_sc=False, needs_layout_passes=False),
    )


def _scatter_body(mw_hbm, ep_hbm, out_hbm, mw_v, ep_v, rho_v, red_v, acc_v, shared):
    cid = lax.axis_index("c")
    sid = lax.axis_index("s")
    wid = sid * NC + cid
    base = wid * EPW
    pltpu.sync_copy(mw_hbm.at[pl.ds(base, EPW)], mw_v)
    pltpu.sync_copy(ep_hbm.at[pl.ds(base, EPW)], ep_v)

    def zero(j, carry):
        rho_v[pl.ds(j * LANES, LANES)] = jnp.zeros((LANES,), jnp.float32)
        return carry

    lax.fori_loop(0, P_PAD // LANES, zero, 0)

    def scat(i, carry):
        idx = ep_v[pl.ds(i * LANES, LANES)]
        val = mw_v[pl.ds(i * LANES, LANES)]
        plsc.addupdate_scatter(rho_v, [idx], val)
        return carry

    lax.fori_loop(0, EPW // LANES, scat, 0)

    pltpu.sync_copy(rho_v, shared.at[sid])
    plsc.subcore_barrier()

    def rd(k, carry):
        pltpu.sync_copy(shared.at[k, pl.ds(sid * SEG, SEG)], red_v.at[k])
        return carry

    lax.fori_loop(0, NS, rd, 0)

    def red(j, carry):
        s = red_v[0, pl.ds(j * LANES, LANES)]
        for k in range(1, NS):
            s = s + red_v[k, pl.ds(j * LANES, LANES)]
        acc_v[pl.ds(j * LANES, LANES)] = s
        return carry

    lax.fori_loop(0, SEG // LANES, red, 0)
    pltpu.sync_copy(acc_v, out_hbm.at[cid, pl.ds(sid * SEG, SEG)])


@functools.lru_cache(maxsize=None)
def _scatter_call():
    return pl.kernel(
        _scatter_body,
        out_type=jax.ShapeDtypeStruct((NC, P_PAD), jnp.float32),
        mesh=plsc.VectorSubcoreMesh(**_MESH),
        scratch_types=[
            pltpu.VMEM((EPW,), jnp.float32),
            pltpu.VMEM((EPW,), jnp.int32),
            pltpu.VMEM((P_PAD,), jnp.float32),
            pltpu.VMEM((NS, SEG), jnp.float32),
            pltpu.VMEM((SEG,), jnp.float32),
            pltpu.VMEM_SHARED((NS, P_PAD), jnp.float32),
        ],
        compiler_params=pltpu.CompilerParams(
            use_tc_tiling_on_sc=False, needs_layout_passes=False),
    )


def _mlp_body(g_ref, x_ref, pk_ref, cmt_ref, w1a_ref, w1b_ref, w1c_ref,
              w1d_ref, b1_ref, w2_ref, b2_ref, w3_ref, b3_ref, out_ref):
    g = g_ref[...]
    s_e = g[:, 0:F_HID]
    v0 = g[:, F_HID:2 * F_HID]
    v1 = g[:, 2 * F_HID:3 * F_HID]
    v2 = g[:, 3 * F_HID:4 * F_HID]
    axt = x_ref[0:3, :]
    pxt = x_ref[3:6, :]
    dispt = pk_ref[0:3, :]
    pft = pk_ref[3:4, :]

    dot = functools.partial(jnp.dot, preferred_element_type=jnp.float32)
    bft = jnp.floor(pft * (1.0 / P_MAX))
    iota4 = lax.broadcasted_iota(jnp.int32, (B, 1), 0).astype(jnp.float32)
    oht = (bft == iota4).astype(jnp.float32)
    dvallt = jnp.dot(cmt_ref[0:3 * B, 0:3], dispt,
                     preferred_element_type=jnp.float32,
                     precision=lax.Precision.HIGHEST)
    dvt = (oht[0:1] * dvallt[0:3] + oht[1:2] * dvallt[3:6] +
           oht[2:3] * dvallt[6:9] + oht[3:4] * dvallt[9:12])
    difft = pxt - (axt + dvt)
    d2t = jnp.sum(difft * difft, axis=0, keepdims=True)
    distt = jnp.sqrt(d2t)
    invt = 1.0 / (distt + 1e-8)
    inv0t = 1.0 / distt
    cwt = 0.5 * (jnp.cos(distt * (jnp.pi / CUTOFF)) + 1.0)
    cwt = jnp.where(distt < CUTOFF, cwt, 0.0)
    nvec = lax.broadcasted_iota(jnp.int32, (D_RBF, 1), 0).astype(
        jnp.float32) + 1.0
    e_pit = jnp.sin(distt * nvec * (jnp.pi / CUTOFF)) * inv0t
    e_pi = jnp.transpose(e_pit)
    rhat = jnp.transpose(difft * invt)
    q_pi = v0 * rhat[:, 0:1] + v1 * rhat[:, 1:2] + v2 * rhat[:, 2:3]
    n_e = jnp.sqrt(v0 * v0 + v1 * v1 + v2 * v2)

    pre1 = (dot(e_pi, w1a_ref[...]) + dot(q_pi, w1b_ref[...]) +
            dot(n_e, w1c_ref[...]) + dot(s_e, w1d_ref[...]) + b1_ref[...])
    h1 = pre1 * jax.nn.sigmoid(pre1)
    pre2 = dot(h1, w2_ref[...]) + b2_ref[...]
    h2 = pre2 * jax.nn.sigmoid(pre2)
    m = jnp.sum(h2 * w3_ref[...], axis=1, keepdims=True) + b3_ref[...]
    out_ref[...] = m * jnp.transpose(cwt)


_mlp = pl.pallas_call(
    _mlp_body,
    grid=(NTB,),
    in_specs=[
        pl.BlockSpec((TBLK, ROW), lambda i: (i, 0)),
        pl.BlockSpec((8, TBLK), lambda i: (0, i)),
        pl.BlockSpec((8, TBLK), lambda i: (0, i)),
        pl.BlockSpec((16, 8), lambda i: (0, 0)),
        pl.BlockSpec((D_RBF, F_HID), lambda i: (0, 0)),
        pl.BlockSpec((F_HID, F_HID), lambda i: (0, 0)),
        pl.BlockSpec((F_HID, F_HID), lambda i: (0, 0)),
        pl.BlockSpec((F_HID, F_HID), lambda i: (0, 0)),
        pl.BlockSpec((1, F_HID), lambda i: (0, 0)),
        pl.BlockSpec((F_HID, F_HID // 2), lambda i: (0, 0)),
        pl.BlockSpec((1, F_HID // 2), lambda i: (0, 0)),
        pl.BlockSpec((1, F_HID // 2), lambda i: (0, 0)),
        pl.BlockSpec((1, 1), lambda i: (0, 0)),
    ],
    out_specs=pl.BlockSpec((TBLK, 1), lambda i: (i, 0)),
    out_shape=jax.ShapeDtypeStruct((E_HALF, 1), jnp.float32),
    compiler_params=pltpu.CompilerParams(
        dimension_semantics=("arbitrary",)),
)


def kernel(atom_xyz, probe_xyz, cell, edge_atom, edge_probe,
           probe_edges_displacement, S_JK, V_JK, W1, b1, W2, b2, W3, b3):
    f32 = jnp.float32
    tbl = jnp.concatenate(
        [S_JK, V_JK[:, 0, :], V_JK[:, 1, :], V_JK[:, 2, :]], axis=1)
    xyz6 = jnp.zeros((6, N_ATOMS), f32)
    xyz6 = xyz6.at[0:3, :].set(jnp.transpose(atom_xyz))
    xyz6 = xyz6.at[3:6, 0:N_PROBES].set(jnp.transpose(probe_xyz))

    npad = E_PAD - N_EDGES
    ea_pad = jnp.concatenate([edge_atom, jnp.zeros((npad,), jnp.int32)])
    ep_pad = jnp.concatenate([edge_probe, jnp.zeros((npad,), jnp.int32)])
    xyzt = _xyz_call()(xyz6, ea_pad, ep_pad)

    packedt = jnp.zeros((8, E_PAD), f32)
    packedt = packedt.at[0:3, 0:N_EDGES].set(
        jnp.transpose(probe_edges_displacement))
    packedt = packedt.at[3, 0:N_EDGES].set(edge_probe.astype(f32))
    cmt = jnp.zeros((16, 8), f32).at[:3 * B, :3].set(
        jnp.transpose(cell, (0, 2, 1)).reshape(3 * B, 3))
    mw_halves = []
    for h in range(NH):
        lo = h * E_HALF
        g_h = _gather_call()(tbl, lax.slice_in_dim(ea_pad, lo, lo + E_HALF))
        mw_h = _mlp(g_h, lax.slice_in_dim(xyzt, lo, lo + E_HALF, axis=1),
                    lax.slice_in_dim(packedt, lo, lo + E_HALF, axis=1), cmt,
                    W1[0:D_RBF], W1[D_RBF:D_RBF + F_HID],
                    W1[D_RBF + F_HID:D_RBF + 2 * F_HID],
                    W1[D_RBF + 2 * F_HID:D_RBF + 3 * F_HID],
                    b1[None, :], W2, b2[None, :],
                    W3.reshape(1, F_HID // 2), b3.reshape(1, 1))
        mw_halves.append(mw_h[:, 0])
    mw = jnp.concatenate(mw_halves)

    ep_scat = jnp.concatenate(
        [edge_probe, jnp.full((npad,), DUMP, jnp.int32)])
    part = _scatter_call()(mw, ep_scat)
    rho = part[0, :N_PROBES] + part[1, :N_PROBES]
    return rho.reshape(B, P_MAX)
